# baseline (speedup 1.0000x reference)
import functools

import jax
import jax.numpy as jnp
from jax import lax
from jax.experimental import pallas as pl
from jax.experimental.pallas import tpu as pltpu
from jax.experimental.pallas import tpu_sc as plsc

VOCAB = 1000000
HIDDEN = 64
BATCH = 4096
SEQ = 200

NC = 2
NS = 16
NW = NC * NS

ROWS_PER_W = BATCH // NW
CHUNK = 4
N_CHUNKS = ROWS_PER_W // CHUNK
IDX_PER_CHUNK = CHUNK * SEQ
GSPLIT = 80
N_GATHERS = IDX_PER_CHUNK // GSPLIT
HREG = HIDDEN // 16


def _sc_pool_kernel(xflat_hbm, table_hbm, out_hbm, idx_v, rows_v, pooled_v, sem):
    wid = lax.axis_index("s") * NC + lax.axis_index("c")
    base_flat = wid * (ROWS_PER_W * SEQ)

    def chunk_body(c, carry):
        flat = base_flat + c * IDX_PER_CHUNK
        pltpu.sync_copy(xflat_hbm.at[pl.ds(flat, IDX_PER_CHUNK)], idx_v)
        cps = [
            pltpu.async_copy(
                table_hbm.at[idx_v.at[pl.ds(i * GSPLIT, GSPLIT)]],
                rows_v.at[pl.ds(i * GSPLIT, GSPLIT)],
                sem,
            )
            for i in range(N_GATHERS)
        ]
        for cp in cps:
            cp.wait()
        for r in range(CHUNK):
            def jbody(j, accs):
                row = r * SEQ + j
                return tuple(
                    accs[k] + rows_v[row, pl.ds(k * 16, 16)] for k in range(HREG)
                )
            accs = lax.fori_loop(
                0, SEQ, jbody,
                tuple(jnp.zeros((16,), jnp.float32) for _ in range(HREG)),
            )
            out_base = (c * CHUNK + r) * HIDDEN
            for k in range(HREG):
                pooled_v[pl.ds(out_base + k * 16, 16)] = accs[k]
        return carry

    lax.fori_loop(0, N_CHUNKS, chunk_body, 0)
    pltpu.sync_copy(
        pooled_v, out_hbm.at[pl.ds(wid * (ROWS_PER_W * HIDDEN), ROWS_PER_W * HIDDEN)]
    )


def _sc_pool(xflat, table):
    mesh = plsc.VectorSubcoreMesh(core_axis_name="c", subcore_axis_name="s")
    k = functools.partial(
        pl.kernel,
        mesh=mesh,
        compiler_params=pltpu.CompilerParams(use_tc_tiling_on_sc=False),
        out_type=jax.ShapeDtypeStruct((BATCH * HIDDEN,), jnp.float32),
        scratch_types=[
            pltpu.VMEM((IDX_PER_CHUNK,), jnp.int32),
            pltpu.VMEM((IDX_PER_CHUNK, HIDDEN), jnp.float32),
            pltpu.VMEM((ROWS_PER_W * HIDDEN,), jnp.float32),
            pltpu.SemaphoreType.DMA,
        ],
    )(_sc_pool_kernel)
    return k(xflat, table)


def _fc_kernel(p_ref, wt_ref, b_ref, o_ref):
    p = p_ref[...] * (1.0 / SEQ)
    acc = jnp.dot(p, wt_ref[...], preferred_element_type=jnp.float32)
    o_ref[...] = jnp.maximum(acc + b_ref[...], 0.0)


def kernel(x, emb_table, W, b):
    xflat = x.astype(jnp.int32).reshape(-1)
    pooled = _sc_pool(xflat, emb_table).reshape(BATCH, HIDDEN)
    out = pl.pallas_call(
        _fc_kernel,
        out_shape=jax.ShapeDtypeStruct((BATCH, HIDDEN), jnp.float32),
    )(pooled, W.T, b.reshape(1, HIDDEN))
    return out
